# Initial kernel scaffold; baseline (speedup 1.0000x reference)
import jax
import jax.numpy as jnp
from jax.experimental import pallas as pl


def kernel(inputs, edge_index, adj_vals, W_gate1, b_gate1, W_cand1, b_cand1, W_gate2, b_gate2, W_cand2, b_cand2):
    raise NotImplementedError("write your pallas kernel here")


# speedup vs baseline: 10.1349x; 10.1349x over previous
import functools

import jax
import jax.numpy as jnp
from jax import lax
from jax.experimental import pallas as pl
from jax.experimental.pallas import tpu as pltpu
from jax.experimental.pallas import tpu_sc as plsc

N = 10000
E = 160000
U = 64
IN = 2
B = 2
NPAD = 10240
EPAD = 163840
ER = 1280
RPT = 80
NPT = 640
RBLK = 2048

_mesh = plsc.VectorSubcoreMesh(core_axis_name="c", subcore_axis_name="s")
_sc_params = pltpu.CompilerParams(needs_layout_passes=False,
                                  use_tc_tiling_on_sc=False)


def _i16(val):
    return jnp.full((16,), val, jnp.int32)


@functools.partial(
    pl.kernel,
    out_type=[
        jax.ShapeDtypeStruct((ER, 128), jnp.float32),
        jax.ShapeDtypeStruct((ER, 128), jnp.float32),
        jax.ShapeDtypeStruct((4, IN, B * NPAD), jnp.float32),
    ],
    mesh=_mesh,
    compiler_params=_sc_params,
    scratch_types=[
        pltpu.VMEM((RPT, 128), jnp.int32),
        pltpu.VMEM((RPT, 128), jnp.int32),
        pltpu.VMEM((RPT, 128), jnp.float32),
        pltpu.VMEM((RPT, 128), jnp.float32),
        pltpu.VMEM((RPT, 128), jnp.float32),
        pltpu.VMEM((NPAD,), jnp.float32),
        pltpu.VMEM((NPAD,), jnp.float32),
        pltpu.VMEM((NPAD,), jnp.float32),
        pltpu.VMEM((NPAD,), jnp.float32),
        pltpu.VMEM((NPAD,), jnp.float32),
        pltpu.VMEM((NPAD,), jnp.float32),
        pltpu.VMEM((128,), jnp.float32),
        pltpu.VMEM((128,), jnp.float32),
        pltpu.VMEM((NPT,), jnp.float32),
        pltpu.VMEM((NPT,), jnp.float32),
        pltpu.VMEM_SHARED((NPAD,), jnp.float32),
        pltpu.VMEM_SHARED((NPAD,), jnp.float32),
        pltpu.VMEM_SHARED((NPAD,), jnp.float32),
        pltpu.VMEM_SHARED((NPAD,), jnp.float32),
        pltpu.SemaphoreType.DMA,
    ],
)
def _l1_kernel(src_h, dst_h, adj_h, x0_h, wn1_h, wn2_h, xs1_h,
               ev_src, ev_dst, ev_adj, ev_w1, ev_w2, inv_s, inv_d,
               xc0, xc1, cur0, cur1, valb0, valb1, zb1, nsl,
               acc_s, acc_d, accL0, accL1, sem):
    c = lax.axis_index("c")
    t = lax.axis_index("s")
    r0 = t * RPT
    n0 = t * NPT
    iota = lax.iota(jnp.int32, 16)
    zv = jnp.zeros((16,), jnp.float32)

    pltpu.sync_copy(src_h.at[pl.ds(r0, RPT)], ev_src)
    pltpu.sync_copy(dst_h.at[pl.ds(r0, RPT)], ev_dst)
    pltpu.sync_copy(adj_h.at[pl.ds(r0, RPT)], ev_adj)
    pltpu.sync_copy(x0_h.at[0].at[pl.ds(c * NPAD, NPAD)], xc0)
    pltpu.sync_copy(x0_h.at[1].at[pl.ds(c * NPAD, NPAD)], xc1)

    @pl.loop(0, NPT // 16)
    def _(k):
        plsc.store_scatter(zb1, [k * 16 + iota], zv)

    for accref in (acc_s, acc_d, accL0, accL1):
        pltpu.sync_copy(zb1, accref.at[pl.ds(n0, NPT)])
    plsc.subcore_barrier()

    @pl.loop(0, RPT // 2)
    def _(i2):
        i0 = i2 * 2
        d0 = pltpu.make_async_copy(ev_adj.at[i0], acc_s.at[ev_src.at[i0]], sem)
        d0.start(add=True)
        d1 = pltpu.make_async_copy(ev_adj.at[i0], acc_d.at[ev_dst.at[i0]], sem)
        d1.start(add=True)
        d2 = pltpu.make_async_copy(ev_adj.at[i0 + 1], acc_s.at[ev_src.at[i0 + 1]], sem)
        d2.start(add=True)
        d3 = pltpu.make_async_copy(ev_adj.at[i0 + 1], acc_d.at[ev_dst.at[i0 + 1]], sem)
        d3.start(add=True)
        d0.wait()
        d1.wait()
        d2.wait()
        d3.wait()
    plsc.subcore_barrier()

    for accref in (acc_s, acc_d):
        pltpu.sync_copy(accref.at[pl.ds(n0, NPT)], nsl)

        @pl.loop(0, NPT // 16)
        def _(k):
            f = k * 16 + iota
            v = plsc.load_gather(nsl, [f])
            v = jnp.where(v > 0.0, v, 1.0)
            plsc.store_scatter(nsl, [f], 1.0 / v)

        pltpu.sync_copy(nsl, accref.at[pl.ds(n0, NPT)])
    plsc.subcore_barrier()
    pltpu.sync_copy(acc_s, inv_s)
    pltpu.sync_copy(acc_d, inv_d)

    @pl.loop(0, RPT)
    def _(i):
        ri = _i16(i)

        @pl.loop(0, 8)
        def _(k):
            col = k * 16 + iota
            av = plsc.load_gather(ev_adj, [ri, col])
            sv = plsc.load_gather(ev_src, [ri, col])
            dv = plsc.load_gather(ev_dst, [ri, col])
            plsc.store_scatter(ev_w1, [ri, col], av * plsc.load_gather(inv_s, [sv]))
            plsc.store_scatter(ev_w2, [ri, col], av * plsc.load_gather(inv_d, [dv]))

    @pl.when(c == 0)
    def _():
        pltpu.sync_copy(ev_w1, wn1_h.at[pl.ds(r0, RPT)])
        pltpu.sync_copy(ev_w2, wn2_h.at[pl.ds(r0, RPT)])

    for s in range(4):
        g0 = xc0 if s in (0, 2) else cur0
        g1 = xc1 if s in (0, 2) else cur1
        idxg = ev_src if s < 2 else ev_dst
        idxs = ev_dst if s < 2 else ev_src
        wb = ev_w1 if s < 2 else ev_w2

        @pl.loop(0, RPT)
        def _(i, idxg=idxg, idxs=idxs, wb=wb, g0=g0, g1=g1):
            ri = _i16(i)

            @pl.loop(0, 8)
            def _(k, idxg=idxg, wb=wb, g0=g0, g1=g1, ri=ri):
                col = k * 16 + iota
                sv = plsc.load_gather(idxg, [ri, col])
                wv = plsc.load_gather(wb, [ri, col])
                plsc.store_scatter(valb0, [col], wv * plsc.load_gather(g0, [sv]))
                plsc.store_scatter(valb1, [col], wv * plsc.load_gather(g1, [sv]))

            da = pltpu.make_async_copy(valb0, accL0.at[idxs.at[i]], sem)
            da.start(add=True)
            db = pltpu.make_async_copy(valb1, accL1.at[idxs.at[i]], sem)
            db.start(add=True)
            da.wait()
            db.wait()

        plsc.subcore_barrier()
        if s in (0, 2):
            pltpu.sync_copy(accL0, cur0)
            pltpu.sync_copy(accL1, cur1)
        for d, accref in ((0, accL0), (1, accL1)):
            pltpu.sync_copy(accref.at[pl.ds(n0, NPT)], nsl)
            pltpu.sync_copy(nsl, xs1_h.at[s].at[d].at[pl.ds(c * NPAD + n0, NPT)])
        plsc.subcore_barrier()
        if s < 3:
            pltpu.sync_copy(zb1, accL0.at[pl.ds(n0, NPT)])
            pltpu.sync_copy(zb1, accL1.at[pl.ds(n0, NPT)])
            plsc.subcore_barrier()


@functools.partial(
    pl.kernel,
    out_type=jax.ShapeDtypeStruct((4, B * NPAD, U), jnp.float32),
    mesh=_mesh,
    compiler_params=_sc_params,
    scratch_types=[
        pltpu.VMEM((RPT, 128), jnp.int32),
        pltpu.VMEM((RPT, 128), jnp.int32),
        pltpu.VMEM((RPT, 128), jnp.int32),
        pltpu.VMEM((RPT, 128), jnp.int32),
        pltpu.VMEM((RPT, 128), jnp.float32),
        pltpu.VMEM((RPT, 128), jnp.float32),
        pltpu.VMEM((128, U), jnp.float32),
        pltpu.VMEM((128, U), jnp.float32),
        pltpu.VMEM_SHARED((NPAD, U), jnp.float32),
        pltpu.SemaphoreType.DMA,
        pltpu.SemaphoreType.DMA,
    ],
)
def _l2_kernel(src_h, dst_h, wn1_h, wn2_h, x_h, xs2_h,
               ev_src, ev_dst, ev_srcg, ev_dstg, ev_w1, ev_w2,
               gb0, gb1, acc2, sem_g, sem_s):

    def _fill_zero_gb0():
        zv16 = jnp.zeros((16,), jnp.float32)
        io = lax.iota(jnp.int32, 16)

        @pl.loop(0, 128 * U // 16)
        def _(k):
            f = k * 16 + io
            plsc.store_scatter(gb0, [f // U, f % U], zv16)
    c = lax.axis_index("c")
    t = lax.axis_index("s")
    r0 = t * RPT
    n0 = t * NPT
    off = c * NPAD
    iota = lax.iota(jnp.int32, 16)
    zv = jnp.zeros((16,), jnp.float32)

    pltpu.sync_copy(src_h.at[pl.ds(r0, RPT)], ev_src)
    pltpu.sync_copy(dst_h.at[pl.ds(r0, RPT)], ev_dst)
    pltpu.sync_copy(wn1_h.at[pl.ds(r0, RPT)], ev_w1)
    pltpu.sync_copy(wn2_h.at[pl.ds(r0, RPT)], ev_w2)

    @pl.loop(0, RPT)
    def _(i):
        ri = _i16(i)

        @pl.loop(0, 8)
        def _(k):
            col = k * 16 + iota
            sv = plsc.load_gather(ev_src, [ri, col])
            dv = plsc.load_gather(ev_dst, [ri, col])
            plsc.store_scatter(ev_srcg, [ri, col], sv + off)
            plsc.store_scatter(ev_dstg, [ri, col], dv + off)

    _fill_zero_gb0()
    for kc in range(NPT // 128):
        pltpu.sync_copy(gb0, acc2.at[pl.ds(n0 + kc * 128, 128)])
    plsc.subcore_barrier()

    for s in range(4):
        gsrc_h = x_h if s in (0, 2) else xs2_h.at[s - 1]
        idxg = ev_srcg if s < 2 else ev_dstg
        idxs = ev_dst if s < 2 else ev_src
        wb = ev_w1 if s < 2 else ev_w2

        pltpu.make_async_copy(gsrc_h.at[idxg.at[0]], gb0, sem_g).start()
        pltpu.make_async_copy(gsrc_h.at[idxg.at[1]], gb1, sem_g).start()

        @pl.loop(0, RPT // 2)
        def _(i2, gsrc_h=gsrc_h, idxg=idxg, idxs=idxs, wb=wb):
            for b, gb in ((0, gb0), (1, gb1)):
                i = i2 * 2 + b
                pltpu.make_async_copy(gsrc_h.at[idxg.at[i]], gb, sem_g).wait()

                @pl.loop(0, 128)
                def _(e, gb=gb, wb=wb, i=i):
                    re = _i16(e)
                    wv = plsc.load_gather(wb, [_i16(i), re])
                    for j in range(U // 16):
                        col = j * 16 + iota
                        v = plsc.load_gather(gb, [re, col])
                        plsc.store_scatter(gb, [re, col], v * wv)

                dsc = pltpu.make_async_copy(gb, acc2.at[idxs.at[i]], sem_s)
                dsc.start(add=True)
                dsc.wait()

                @pl.when(i + 2 < RPT)
                def _(gsrc_h=gsrc_h, idxg=idxg, gb=gb, i=i):
                    pltpu.make_async_copy(gsrc_h.at[idxg.at[i + 2]], gb, sem_g).start()

        plsc.subcore_barrier()
        for kc in range(NPT // 128):
            rows = pl.ds(n0 + kc * 128, 128)
            pltpu.sync_copy(acc2.at[rows], gb1)
            pltpu.sync_copy(gb1, xs2_h.at[s].at[pl.ds(off + n0 + kc * 128, 128)])
        plsc.subcore_barrier()
        if s < 3:
            _fill_zero_gb0()
            for kc in range(NPT // 128):
                pltpu.sync_copy(gb0, acc2.at[pl.ds(n0 + kc * 128, 128)])
            plsc.subcore_barrier()


def _gate_tail(acc, o_ref):
    u = jax.nn.sigmoid(acc[:, :U])
    cc = jnp.tanh(acc[:, U:])
    o_ref[...] = (1.0 - u) * cc


def _tc1_body(x0_ref, xs_ref, w_ref, b_ref, o_ref):
    acc = jnp.zeros((RBLK, 2 * U), jnp.float32) + b_ref[...]
    for m in range(5):
        for d in range(IN):
            col = x0_ref[d] if m == 0 else xs_ref[m - 1, d]
            acc = acc + col[:, None] * w_ref[m, d][None, :]
    _gate_tail(acc, o_ref)


def _tc1_call(x0c, xs1, wstack, bias):
    grid = (B * NPAD) // RBLK
    return pl.pallas_call(
        _tc1_body,
        grid=(grid,),
        in_specs=[
            pl.BlockSpec((IN, RBLK), lambda i: (0, i)),
            pl.BlockSpec((4, IN, RBLK), lambda i: (0, 0, i)),
            pl.BlockSpec((5, IN, 2 * U), lambda i: (0, 0, 0)),
            pl.BlockSpec((1, 2 * U), lambda i: (0, 0)),
        ],
        out_specs=pl.BlockSpec((RBLK, U), lambda i: (i, 0)),
        out_shape=jax.ShapeDtypeStruct((B * NPAD, U), jnp.float32),
    )(x0c, xs1, wstack, bias)


def _tc2_body(x0_ref, xs_ref, w_ref, b_ref, o_ref):
    acc = jnp.dot(x0_ref[...], w_ref[0], preferred_element_type=jnp.float32)
    for m in range(4):
        acc = acc + jnp.dot(xs_ref[m], w_ref[m + 1],
                            preferred_element_type=jnp.float32)
    acc = acc + b_ref[...]
    _gate_tail(acc, o_ref)


def _tc2_call(x0, xs, wstack, bias):
    grid = (B * NPAD) // RBLK
    return pl.pallas_call(
        _tc2_body,
        grid=(grid,),
        in_specs=[
            pl.BlockSpec((RBLK, U), lambda i: (i, 0)),
            pl.BlockSpec((4, RBLK, U), lambda i: (0, i, 0)),
            pl.BlockSpec((5, U, 2 * U), lambda i: (0, 0, 0)),
            pl.BlockSpec((1, 2 * U), lambda i: (0, 0)),
        ],
        out_specs=pl.BlockSpec((RBLK, U), lambda i: (i, 0)),
        out_shape=jax.ShapeDtypeStruct((B * NPAD, U), jnp.float32),
    )(x0, xs, wstack, bias)


def _prep_weights(Wg, bg, Wc, bc, din):
    dfull = Wg.shape[0] // 5
    Wgr = Wg.reshape(dfull, 5, 2 * U)[:din, :, U:]
    Wcr = Wc.reshape(dfull, 5, U)[:din, :, :]
    Wm = jnp.concatenate([Wgr, Wcr], axis=-1)
    Wt = jnp.moveaxis(Wm, 1, 0)
    W0 = Wt[0] - Wt[2] - Wt[4]
    Ws = jnp.stack([W0, Wt[1], 2.0 * Wt[2], Wt[3], 2.0 * Wt[4]], 0)
    bias = jnp.concatenate([bg[U:], bc])[None, :]
    return Ws, bias


def kernel(inputs, edge_index, adj_vals, W_gate1, b_gate1, W_cand1, b_cand1,
           W_gate2, b_gate2, W_cand2, b_cand2):
    src = edge_index[0]
    dst = edge_index[1]
    pe = EPAD - E
    srcp = jnp.concatenate([src, jnp.zeros((pe,), jnp.int32)]).reshape(ER, 128)
    dstp = jnp.concatenate([dst, jnp.zeros((pe,), jnp.int32)]).reshape(ER, 128)
    adjp = jnp.concatenate([adj_vals, jnp.zeros((pe,), jnp.float32)]).reshape(ER, 128)
    x0 = inputs.reshape(B, N, IN)
    x0p = jnp.pad(x0, ((0, 0), (0, NPAD - N), (0, 0)))
    x0c = x0p.transpose(2, 0, 1).reshape(IN, B * NPAD)

    wn1, wn2, xs1 = _l1_kernel(srcp, dstp, adjp, x0c)
    Ws1, b1 = _prep_weights(W_gate1, b_gate1, W_cand1, b_cand1, IN)
    nh1p = _tc1_call(x0c, xs1, Ws1, b1)

    xs2 = _l2_kernel(srcp, dstp, wn1, wn2, nh1p)
    Ws2, b2 = _prep_weights(W_gate2, b_gate2, W_cand2, b_cand2, U)
    nh2p = _tc2_call(nh1p, xs2, Ws2, b2)

    def unpad(a):
        return a.reshape(B, NPAD, U)[:, :N, :].reshape(B, N * U)

    h1 = unpad(nh1p)
    h2 = unpad(nh2p)
    return h2, jnp.stack([h1, h2], 0)


# speedup vs baseline: 11.0572x; 1.0910x over previous
import functools

import jax
import jax.numpy as jnp
from jax import lax
from jax.experimental import pallas as pl
from jax.experimental.pallas import tpu as pltpu
from jax.experimental.pallas import tpu_sc as plsc

N = 10000
E = 160000
U = 64
IN = 2
B = 2
NPAD = 10240
EPAD = 163840
ER = 1280
RPT = 80
NPT = 640
RBLK = 2048

_mesh = plsc.VectorSubcoreMesh(core_axis_name="c", subcore_axis_name="s")
_sc_params = pltpu.CompilerParams(needs_layout_passes=False,
                                  use_tc_tiling_on_sc=False)


def _i16(val):
    return jnp.full((16,), val, jnp.int32)


@functools.partial(
    pl.kernel,
    out_type=[
        jax.ShapeDtypeStruct((ER, 128), jnp.float32),
        jax.ShapeDtypeStruct((ER, 128), jnp.float32),
        jax.ShapeDtypeStruct((4, IN, B * NPAD), jnp.float32),
    ],
    mesh=_mesh,
    compiler_params=_sc_params,
    scratch_types=[
        pltpu.VMEM((RPT, 128), jnp.int32),
        pltpu.VMEM((RPT, 128), jnp.int32),
        pltpu.VMEM((RPT, 128), jnp.float32),
        pltpu.VMEM((RPT, 128), jnp.float32),
        pltpu.VMEM((RPT, 128), jnp.float32),
        pltpu.VMEM((NPAD,), jnp.float32),
        pltpu.VMEM((NPAD,), jnp.float32),
        pltpu.VMEM((NPAD,), jnp.float32),
        pltpu.VMEM((NPAD,), jnp.float32),
        pltpu.VMEM((NPAD,), jnp.float32),
        pltpu.VMEM((NPAD,), jnp.float32),
        pltpu.VMEM((128,), jnp.float32),
        pltpu.VMEM((128,), jnp.float32),
        pltpu.VMEM((NPT,), jnp.float32),
        pltpu.VMEM((NPT,), jnp.float32),
        pltpu.VMEM_SHARED((NPAD,), jnp.float32),
        pltpu.VMEM_SHARED((NPAD,), jnp.float32),
        pltpu.VMEM_SHARED((NPAD,), jnp.float32),
        pltpu.VMEM_SHARED((NPAD,), jnp.float32),
        pltpu.SemaphoreType.DMA,
    ],
)
def _l1_kernel(src_h, dst_h, adj_h, x0_h, wn1_h, wn2_h, xs1_h,
               ev_src, ev_dst, ev_adj, ev_w1, ev_w2, inv_s, inv_d,
               xc0, xc1, cur0, cur1, valb0, valb1, zb1, nsl,
               acc_s, acc_d, accL0, accL1, sem):
    c = lax.axis_index("c")
    t = lax.axis_index("s")
    r0 = t * RPT
    n0 = t * NPT
    iota = lax.iota(jnp.int32, 16)
    zv = jnp.zeros((16,), jnp.float32)

    pltpu.sync_copy(src_h.at[pl.ds(r0, RPT)], ev_src)
    pltpu.sync_copy(dst_h.at[pl.ds(r0, RPT)], ev_dst)
    pltpu.sync_copy(adj_h.at[pl.ds(r0, RPT)], ev_adj)
    pltpu.sync_copy(x0_h.at[0].at[pl.ds(c * NPAD, NPAD)], xc0)
    pltpu.sync_copy(x0_h.at[1].at[pl.ds(c * NPAD, NPAD)], xc1)

    @pl.loop(0, NPT // 16)
    def _(k):
        plsc.store_scatter(zb1, [k * 16 + iota], zv)

    for accref in (acc_s, acc_d, accL0, accL1):
        pltpu.sync_copy(zb1, accref.at[pl.ds(n0, NPT)])
    plsc.subcore_barrier()

    @pl.loop(0, RPT // 2)
    def _(i2):
        i0 = i2 * 2
        d0 = pltpu.make_async_copy(ev_adj.at[i0], acc_s.at[ev_src.at[i0]], sem)
        d0.start(add=True)
        d1 = pltpu.make_async_copy(ev_adj.at[i0], acc_d.at[ev_dst.at[i0]], sem)
        d1.start(add=True)
        d2 = pltpu.make_async_copy(ev_adj.at[i0 + 1], acc_s.at[ev_src.at[i0 + 1]], sem)
        d2.start(add=True)
        d3 = pltpu.make_async_copy(ev_adj.at[i0 + 1], acc_d.at[ev_dst.at[i0 + 1]], sem)
        d3.start(add=True)
        d0.wait()
        d1.wait()
        d2.wait()
        d3.wait()
    plsc.subcore_barrier()

    for accref in (acc_s, acc_d):
        pltpu.sync_copy(accref.at[pl.ds(n0, NPT)], nsl)

        @pl.loop(0, NPT // 16)
        def _(k):
            f = k * 16 + iota
            v = plsc.load_gather(nsl, [f])
            v = jnp.where(v > 0.0, v, 1.0)
            plsc.store_scatter(nsl, [f], 1.0 / v)

        pltpu.sync_copy(nsl, accref.at[pl.ds(n0, NPT)])
    plsc.subcore_barrier()
    pltpu.sync_copy(acc_s, inv_s)
    pltpu.sync_copy(acc_d, inv_d)

    @pl.loop(0, RPT)
    def _(i):
        ri = _i16(i)

        @pl.loop(0, 8)
        def _(k):
            col = k * 16 + iota
            av = plsc.load_gather(ev_adj, [ri, col])
            sv = plsc.load_gather(ev_src, [ri, col])
            dv = plsc.load_gather(ev_dst, [ri, col])
            plsc.store_scatter(ev_w1, [ri, col], av * plsc.load_gather(inv_s, [sv]))
            plsc.store_scatter(ev_w2, [ri, col], av * plsc.load_gather(inv_d, [dv]))

    @pl.when(c == 0)
    def _():
        pltpu.sync_copy(ev_w1, wn1_h.at[pl.ds(r0, RPT)])
        pltpu.sync_copy(ev_w2, wn2_h.at[pl.ds(r0, RPT)])

    for s in range(4):
        g0 = xc0 if s in (0, 2) else cur0
        g1 = xc1 if s in (0, 2) else cur1
        idxg = ev_src if s < 2 else ev_dst
        idxs = ev_dst if s < 2 else ev_src
        wb = ev_w1 if s < 2 else ev_w2

        @pl.loop(0, RPT)
        def _(i, idxg=idxg, idxs=idxs, wb=wb, g0=g0, g1=g1):
            ri = _i16(i)

            @pl.loop(0, 8)
            def _(k, idxg=idxg, wb=wb, g0=g0, g1=g1, ri=ri):
                col = k * 16 + iota
                sv = plsc.load_gather(idxg, [ri, col])
                wv = plsc.load_gather(wb, [ri, col])
                plsc.store_scatter(valb0, [col], wv * plsc.load_gather(g0, [sv]))
                plsc.store_scatter(valb1, [col], wv * plsc.load_gather(g1, [sv]))

            da = pltpu.make_async_copy(valb0, accL0.at[idxs.at[i]], sem)
            da.start(add=True)
            db = pltpu.make_async_copy(valb1, accL1.at[idxs.at[i]], sem)
            db.start(add=True)
            da.wait()
            db.wait()

        plsc.subcore_barrier()
        if s in (0, 2):
            pltpu.sync_copy(accL0, cur0)
            pltpu.sync_copy(accL1, cur1)
        for d, accref in ((0, accL0), (1, accL1)):
            pltpu.sync_copy(accref.at[pl.ds(n0, NPT)], nsl)
            pltpu.sync_copy(nsl, xs1_h.at[s].at[d].at[pl.ds(c * NPAD + n0, NPT)])
        plsc.subcore_barrier()
        if s < 3:
            pltpu.sync_copy(zb1, accL0.at[pl.ds(n0, NPT)])
            pltpu.sync_copy(zb1, accL1.at[pl.ds(n0, NPT)])
            plsc.subcore_barrier()


@functools.partial(
    pl.kernel,
    out_type=jax.ShapeDtypeStruct((4, B * NPAD, U), jnp.float32),
    mesh=_mesh,
    compiler_params=_sc_params,
    scratch_types=[
        pltpu.VMEM((RPT, 128), jnp.int32),
        pltpu.VMEM((RPT, 128), jnp.int32),
        pltpu.VMEM((RPT, 128), jnp.float32),
        pltpu.VMEM((RPT, 128), jnp.float32),
        [pltpu.VMEM((128, U), jnp.float32) for _ in range(4)],
        [pltpu.VMEM((128, U), jnp.float32) for _ in range(2)],
        pltpu.VMEM_SHARED((NPAD, U), jnp.float32),
        pltpu.SemaphoreType.DMA,
        pltpu.SemaphoreType.DMA,
    ],
)
def _l2_kernel(src_h, dst_h, wn1_h, wn2_h, x_h, xs2_h,
               ev_src, ev_dst, ev_w1, ev_w2, gbs, sbs, acc2, sem_g, sem_s):
    c = lax.axis_index("c")
    t = lax.axis_index("s")
    r0 = t * RPT
    n0 = t * NPT
    off = c * NPAD
    iota = lax.iota(jnp.int32, 16)
    zv = jnp.zeros((16,), jnp.float32)

    def _fill_zero(gb):
        @pl.loop(0, 128 * U // 16)
        def _(k, gb=gb):
            f = k * 16 + iota
            plsc.store_scatter(gb, [f // U, f % U], zv)

    pltpu.sync_copy(src_h.at[pl.ds(r0, RPT)], ev_src)
    pltpu.sync_copy(dst_h.at[pl.ds(r0, RPT)], ev_dst)
    pltpu.sync_copy(wn1_h.at[pl.ds(r0, RPT)], ev_w1)
    pltpu.sync_copy(wn2_h.at[pl.ds(r0, RPT)], ev_w2)

    _fill_zero(gbs[0])
    for kc in range(NPT // 128):
        pltpu.sync_copy(gbs[0], acc2.at[pl.ds(n0 + kc * 128, 128)])
    plsc.subcore_barrier()

    for s in range(4):
        gfull = x_h if s in (0, 2) else xs2_h.at[s - 1]
        gsrc_h = gfull.at[pl.ds(off, NPAD)]
        idxg = ev_src if s < 2 else ev_dst
        idxs = ev_dst if s < 2 else ev_src
        wb = ev_w1 if s < 2 else ev_w2

        def _gather(i, gb, gsrc_h=gsrc_h, idxg=idxg):
            return pltpu.make_async_copy(gsrc_h.at[idxg.at[i]], gb, sem_g)

        def _scatter(i, sb, idxs=idxs):
            return pltpu.make_async_copy(sb, acc2.at[idxs.at[i]], sem_s)

        for b in range(2):
            _gather(b, gbs[b]).start()

        @pl.loop(0, RPT // 4)
        def _(i4, wb=wb, _gather=_gather, _scatter=_scatter):
            for b in range(4):
                i = i4 * 4 + b
                gb = gbs[b]
                sb = sbs[b % 2]
                _gather(i, gb).wait()

                prev = i - 2

                @pl.when(prev >= 0)
                def _(prev=prev, sb=sb, _scatter=_scatter):
                    _scatter(prev, sb).wait()

                @pl.loop(0, 128, unroll=8)
                def _(e, gb=gb, sb=sb, wb=wb, i=i):
                    re = _i16(e)
                    wv = plsc.load_gather(wb, [_i16(i), re])
                    for j in range(U // 16):
                        col = j * 16 + iota
                        v = plsc.load_gather(gb, [re, col])
                        plsc.store_scatter(sb, [re, col], v * wv)

                _scatter(i, sb).start(add=True)
                nxt = i + 2

                @pl.when(nxt < RPT)
                def _(nxt=nxt, nb=(b + 2) % 4, _gather=_gather):
                    _gather(nxt, gbs[nb]).start()

        _scatter(RPT - 2, sbs[0]).wait()
        _scatter(RPT - 1, sbs[1]).wait()
        plsc.subcore_barrier()
        pltpu.sync_copy(acc2.at[pl.ds(n0, NPT)],
                        xs2_h.at[s].at[pl.ds(off + n0, NPT)])
        plsc.subcore_barrier()
        if s < 3:
            _fill_zero(gbs[0])
            for kc in range(NPT // 128):
                pltpu.sync_copy(gbs[0], acc2.at[pl.ds(n0 + kc * 128, 128)])
            plsc.subcore_barrier()


def _gate_tail(acc, o_ref):
    u = jax.nn.sigmoid(acc[:, :U])
    cc = jnp.tanh(acc[:, U:])
    o_ref[...] = (1.0 - u) * cc


def _tc1_body(x0_ref, xs_ref, w_ref, b_ref, o_ref):
    acc = jnp.zeros((RBLK, 2 * U), jnp.float32) + b_ref[...]
    for m in range(5):
        for d in range(IN):
            col = x0_ref[d] if m == 0 else xs_ref[m - 1, d]
            acc = acc + col[:, None] * w_ref[m, d][None, :]
    _gate_tail(acc, o_ref)


def _tc1_call(x0c, xs1, wstack, bias):
    grid = (B * NPAD) // RBLK
    return pl.pallas_call(
        _tc1_body,
        grid=(grid,),
        in_specs=[
            pl.BlockSpec((IN, RBLK), lambda i: (0, i)),
            pl.BlockSpec((4, IN, RBLK), lambda i: (0, 0, i)),
            pl.BlockSpec((5, IN, 2 * U), lambda i: (0, 0, 0)),
            pl.BlockSpec((1, 2 * U), lambda i: (0, 0)),
        ],
        out_specs=pl.BlockSpec((RBLK, U), lambda i: (i, 0)),
        out_shape=jax.ShapeDtypeStruct((B * NPAD, U), jnp.float32),
    )(x0c, xs1, wstack, bias)


def _tc2_body(x0_ref, xs_ref, w_ref, b_ref, o_ref):
    acc = jnp.dot(x0_ref[...], w_ref[0], preferred_element_type=jnp.float32)
    for m in range(4):
        acc = acc + jnp.dot(xs_ref[m], w_ref[m + 1],
                            preferred_element_type=jnp.float32)
    acc = acc + b_ref[...]
    _gate_tail(acc, o_ref)


def _tc2_call(x0, xs, wstack, bias):
    grid = (B * NPAD) // RBLK
    return pl.pallas_call(
        _tc2_body,
        grid=(grid,),
        in_specs=[
            pl.BlockSpec((RBLK, U), lambda i: (i, 0)),
            pl.BlockSpec((4, RBLK, U), lambda i: (0, i, 0)),
            pl.BlockSpec((5, U, 2 * U), lambda i: (0, 0, 0)),
            pl.BlockSpec((1, 2 * U), lambda i: (0, 0)),
        ],
        out_specs=pl.BlockSpec((RBLK, U), lambda i: (i, 0)),
        out_shape=jax.ShapeDtypeStruct((B * NPAD, U), jnp.float32),
    )(x0, xs, wstack, bias)


def _prep_weights(Wg, bg, Wc, bc, din):
    dfull = Wg.shape[0] // 5
    Wgr = Wg.reshape(dfull, 5, 2 * U)[:din, :, U:]
    Wcr = Wc.reshape(dfull, 5, U)[:din, :, :]
    Wm = jnp.concatenate([Wgr, Wcr], axis=-1)
    Wt = jnp.moveaxis(Wm, 1, 0)
    W0 = Wt[0] - Wt[2] - Wt[4]
    Ws = jnp.stack([W0, Wt[1], 2.0 * Wt[2], Wt[3], 2.0 * Wt[4]], 0)
    bias = jnp.concatenate([bg[U:], bc])[None, :]
    return Ws, bias


def kernel(inputs, edge_index, adj_vals, W_gate1, b_gate1, W_cand1, b_cand1,
           W_gate2, b_gate2, W_cand2, b_cand2):
    src = edge_index[0]
    dst = edge_index[1]
    pe = EPAD - E
    srcp = jnp.concatenate([src, jnp.zeros((pe,), jnp.int32)]).reshape(ER, 128)
    dstp = jnp.concatenate([dst, jnp.zeros((pe,), jnp.int32)]).reshape(ER, 128)
    adjp = jnp.concatenate([adj_vals, jnp.zeros((pe,), jnp.float32)]).reshape(ER, 128)
    x0 = inputs.reshape(B, N, IN)
    x0p = jnp.pad(x0, ((0, 0), (0, NPAD - N), (0, 0)))
    x0c = x0p.transpose(2, 0, 1).reshape(IN, B * NPAD)

    wn1, wn2, xs1 = _l1_kernel(srcp, dstp, adjp, x0c)
    Ws1, b1 = _prep_weights(W_gate1, b_gate1, W_cand1, b_cand1, IN)
    nh1p = _tc1_call(x0c, xs1, Ws1, b1)

    xs2 = _l2_kernel(srcp, dstp, wn1, wn2, nh1p)
    Ws2, b2 = _prep_weights(W_gate2, b_gate2, W_cand2, b_cand2, U)
    nh2p = _tc2_call(nh1p, xs2, Ws2, b2)

    def unpad(a):
        return a.reshape(B, NPAD, U)[:, :N, :].reshape(B, N * U)

    h1 = unpad(nh1p)
    h2 = unpad(nh2p)
    return h2, jnp.stack([h1, h2], 0)


# speedup vs baseline: 11.0778x; 1.0019x over previous
import functools

import jax
import jax.numpy as jnp
from jax import lax
from jax.experimental import pallas as pl
from jax.experimental.pallas import tpu as pltpu
from jax.experimental.pallas import tpu_sc as plsc

N = 10000
E = 160000
U = 64
IN = 2
B = 2
NPAD = 10240
EPAD = 163840
ER = 1280
RPT = 80
NPT = 640
RBLK = 2048

_mesh = plsc.VectorSubcoreMesh(core_axis_name="c", subcore_axis_name="s")
_sc_params = pltpu.CompilerParams(needs_layout_passes=False,
                                  use_tc_tiling_on_sc=False)


def _i16(val):
    return jnp.full((16,), val, jnp.int32)


@functools.partial(
    pl.kernel,
    out_type=[
        jax.ShapeDtypeStruct((ER, 128), jnp.float32),
        jax.ShapeDtypeStruct((ER, 128), jnp.float32),
        jax.ShapeDtypeStruct((4, IN, B * NPAD), jnp.float32),
    ],
    mesh=_mesh,
    compiler_params=_sc_params,
    scratch_types=[
        pltpu.VMEM((RPT, 128), jnp.int32),
        pltpu.VMEM((RPT, 128), jnp.int32),
        pltpu.VMEM((RPT, 128), jnp.float32),
        pltpu.VMEM((RPT, 128), jnp.float32),
        pltpu.VMEM((RPT, 128), jnp.float32),
        pltpu.VMEM((NPAD,), jnp.float32),
        pltpu.VMEM((NPAD,), jnp.float32),
        pltpu.VMEM((NPAD,), jnp.float32),
        pltpu.VMEM((NPAD,), jnp.float32),
        pltpu.VMEM((NPAD,), jnp.float32),
        pltpu.VMEM((NPAD,), jnp.float32),
        pltpu.VMEM((128,), jnp.float32),
        pltpu.VMEM((128,), jnp.float32),
        pltpu.VMEM((NPT,), jnp.float32),
        pltpu.VMEM((NPT,), jnp.float32),
        pltpu.VMEM_SHARED((NPAD,), jnp.float32),
        pltpu.VMEM_SHARED((NPAD,), jnp.float32),
        pltpu.VMEM_SHARED((NPAD,), jnp.float32),
        pltpu.VMEM_SHARED((NPAD,), jnp.float32),
        pltpu.SemaphoreType.DMA,
    ],
)
def _l1_kernel(src_h, dst_h, adj_h, x0_h, wn1_h, wn2_h, xs1_h,
               ev_src, ev_dst, ev_adj, ev_w1, ev_w2, inv_s, inv_d,
               xc0, xc1, cur0, cur1, valb0, valb1, zb1, nsl,
               acc_s, acc_d, accL0, accL1, sem):
    c = lax.axis_index("c")
    t = lax.axis_index("s")
    r0 = t * RPT
    n0 = t * NPT
    iota = lax.iota(jnp.int32, 16)
    zv = jnp.zeros((16,), jnp.float32)

    pltpu.sync_copy(src_h.at[pl.ds(r0, RPT)], ev_src)
    pltpu.sync_copy(dst_h.at[pl.ds(r0, RPT)], ev_dst)
    pltpu.sync_copy(adj_h.at[pl.ds(r0, RPT)], ev_adj)
    pltpu.sync_copy(x0_h.at[0].at[pl.ds(c * NPAD, NPAD)], xc0)
    pltpu.sync_copy(x0_h.at[1].at[pl.ds(c * NPAD, NPAD)], xc1)

    @pl.loop(0, NPT // 16)
    def _(k):
        plsc.store_scatter(zb1, [k * 16 + iota], zv)

    for accref in (acc_s, acc_d, accL0, accL1):
        pltpu.sync_copy(zb1, accref.at[pl.ds(n0, NPT)])
    plsc.subcore_barrier()

    @pl.loop(0, RPT // 2)
    def _(i2):
        i0 = i2 * 2
        d0 = pltpu.make_async_copy(ev_adj.at[i0], acc_s.at[ev_src.at[i0]], sem)
        d0.start(add=True)
        d1 = pltpu.make_async_copy(ev_adj.at[i0], acc_d.at[ev_dst.at[i0]], sem)
        d1.start(add=True)
        d2 = pltpu.make_async_copy(ev_adj.at[i0 + 1], acc_s.at[ev_src.at[i0 + 1]], sem)
        d2.start(add=True)
        d3 = pltpu.make_async_copy(ev_adj.at[i0 + 1], acc_d.at[ev_dst.at[i0 + 1]], sem)
        d3.start(add=True)
        d0.wait()
        d1.wait()
        d2.wait()
        d3.wait()
    plsc.subcore_barrier()

    for accref in (acc_s, acc_d):
        pltpu.sync_copy(accref.at[pl.ds(n0, NPT)], nsl)

        @pl.loop(0, NPT // 16)
        def _(k):
            f = k * 16 + iota
            v = plsc.load_gather(nsl, [f])
            v = jnp.where(v > 0.0, v, 1.0)
            plsc.store_scatter(nsl, [f], 1.0 / v)

        pltpu.sync_copy(nsl, accref.at[pl.ds(n0, NPT)])
    plsc.subcore_barrier()
    pltpu.sync_copy(acc_s, inv_s)
    pltpu.sync_copy(acc_d, inv_d)

    @pl.loop(0, RPT)
    def _(i):
        ri = _i16(i)

        @pl.loop(0, 8)
        def _(k):
            col = k * 16 + iota
            av = plsc.load_gather(ev_adj, [ri, col])
            sv = plsc.load_gather(ev_src, [ri, col])
            dv = plsc.load_gather(ev_dst, [ri, col])
            plsc.store_scatter(ev_w1, [ri, col], av * plsc.load_gather(inv_s, [sv]))
            plsc.store_scatter(ev_w2, [ri, col], av * plsc.load_gather(inv_d, [dv]))

    @pl.when(c == 0)
    def _():
        pltpu.sync_copy(ev_w1, wn1_h.at[pl.ds(r0, RPT)])
        pltpu.sync_copy(ev_w2, wn2_h.at[pl.ds(r0, RPT)])

    for s in range(4):
        g0 = xc0 if s in (0, 2) else cur0
        g1 = xc1 if s in (0, 2) else cur1
        idxg = ev_src if s < 2 else ev_dst
        idxs = ev_dst if s < 2 else ev_src
        wb = ev_w1 if s < 2 else ev_w2

        @pl.loop(0, RPT)
        def _(i, idxg=idxg, idxs=idxs, wb=wb, g0=g0, g1=g1):
            ri = _i16(i)

            @pl.loop(0, 8)
            def _(k, idxg=idxg, wb=wb, g0=g0, g1=g1, ri=ri):
                col = k * 16 + iota
                sv = plsc.load_gather(idxg, [ri, col])
                wv = plsc.load_gather(wb, [ri, col])
                plsc.store_scatter(valb0, [col], wv * plsc.load_gather(g0, [sv]))
                plsc.store_scatter(valb1, [col], wv * plsc.load_gather(g1, [sv]))

            da = pltpu.make_async_copy(valb0, accL0.at[idxs.at[i]], sem)
            da.start(add=True)
            db = pltpu.make_async_copy(valb1, accL1.at[idxs.at[i]], sem)
            db.start(add=True)
            da.wait()
            db.wait()

        plsc.subcore_barrier()
        if s in (0, 2):
            pltpu.sync_copy(accL0, cur0)
            pltpu.sync_copy(accL1, cur1)
        for d, accref in ((0, accL0), (1, accL1)):
            pltpu.sync_copy(accref.at[pl.ds(n0, NPT)], nsl)
            pltpu.sync_copy(nsl, xs1_h.at[s].at[d].at[pl.ds(c * NPAD + n0, NPT)])
        plsc.subcore_barrier()
        if s < 3:
            pltpu.sync_copy(zb1, accL0.at[pl.ds(n0, NPT)])
            pltpu.sync_copy(zb1, accL1.at[pl.ds(n0, NPT)])
            plsc.subcore_barrier()


UH = U // 2


@functools.partial(
    pl.kernel,
    out_type=jax.ShapeDtypeStruct((4, B * NPAD, U), jnp.float32),
    mesh=_mesh,
    compiler_params=_sc_params,
    scratch_types=[
        pltpu.VMEM((RPT, 128), jnp.int32),
        pltpu.VMEM((RPT, 128), jnp.int32),
        pltpu.VMEM((RPT, 128), jnp.float32),
        pltpu.VMEM((RPT, 128), jnp.float32),
        [pltpu.VMEM((128, UH), jnp.float32) for _ in range(4)],
        [pltpu.VMEM((128, UH), jnp.float32) for _ in range(2)],
        pltpu.VMEM_SHARED((NPAD, UH), jnp.float32),
        pltpu.VMEM_SHARED((NPAD, UH), jnp.float32),
        pltpu.SemaphoreType.DMA,
        pltpu.SemaphoreType.DMA,
    ],
)
def _l2_kernel(src_h, dst_h, wn1_h, wn2_h, x2_h, xs2_h,
               ev_src, ev_dst, ev_w1, ev_w2, gbs, sbs, accA, accB,
               sem_g, sem_s):
    c = lax.axis_index("c")
    t = lax.axis_index("s")
    r0 = t * RPT
    n0 = t * NPT
    off = c * NPAD
    iota = lax.iota(jnp.int32, 16)
    zv = jnp.zeros((16,), jnp.float32)

    def _fill_zero(gb):
        @pl.loop(0, 128 * UH // 16)
        def _(k, gb=gb):
            f = k * 16 + iota
            plsc.store_scatter(gb, [f // UH, f % UH], zv)

    def _zero_accs():
        _fill_zero(gbs[0])
        for accref in (accA, accB):
            for kc in range(NPT // 128):
                pltpu.sync_copy(gbs[0], accref.at[pl.ds(n0 + kc * 128, 128)])

    pltpu.sync_copy(src_h.at[pl.ds(r0, RPT)], ev_src)
    pltpu.sync_copy(dst_h.at[pl.ds(r0, RPT)], ev_dst)
    pltpu.sync_copy(wn1_h.at[pl.ds(r0, RPT)], ev_w1)
    pltpu.sync_copy(wn2_h.at[pl.ds(r0, RPT)], ev_w2)

    _zero_accs()
    plsc.subcore_barrier()

    for h in range(2):
        for s in range(4):
            hbm_src = s in (0, 2)
            gsrc = x2_h.at[h].at[pl.ds(off, NPAD)] if hbm_src else accA
            acc = accA if hbm_src else accB
            idxg = ev_src if s < 2 else ev_dst
            idxs = ev_dst if s < 2 else ev_src
            wb = ev_w1 if s < 2 else ev_w2

            def _gather(i, gb, gsrc=gsrc, idxg=idxg):
                return pltpu.make_async_copy(gsrc.at[idxg.at[i]], gb, sem_g)

            def _scatter(i, sb, idxs=idxs, acc=acc):
                return pltpu.make_async_copy(sb, acc.at[idxs.at[i]], sem_s)

            for b in range(2):
                _gather(b, gbs[b]).start()

            @pl.loop(0, RPT // 4)
            def _(i4, wb=wb, _gather=_gather, _scatter=_scatter):
                for b in range(4):
                    i = i4 * 4 + b
                    gb = gbs[b]
                    sb = sbs[b % 2]
                    _gather(i, gb).wait()

                    prev = i - 2

                    @pl.when(prev >= 0)
                    def _(prev=prev, sb=sb, _scatter=_scatter):
                        _scatter(prev, sb).wait()

                    @pl.loop(0, 128, unroll=8)
                    def _(e, gb=gb, sb=sb, wb=wb, i=i):
                        re = _i16(e)
                        wv = plsc.load_gather(wb, [_i16(i), re])
                        for j in range(UH // 16):
                            col = j * 16 + iota
                            v = plsc.load_gather(gb, [re, col])
                            plsc.store_scatter(sb, [re, col], v * wv)

                    _scatter(i, sb).start(add=True)
                    nxt = i + 2

                    @pl.when(nxt < RPT)
                    def _(nxt=nxt, nb=(b + 2) % 4, _gather=_gather):
                        _gather(nxt, gbs[nb]).start()

            _scatter(RPT - 2, sbs[0]).wait()
            _scatter(RPT - 1, sbs[1]).wait()
            plsc.subcore_barrier()
            pltpu.sync_copy(
                acc.at[pl.ds(n0, NPT)],
                xs2_h.at[s].at[pl.ds(off + n0, NPT), pl.ds(h * UH, UH)])
            if s == 1 or (s == 3 and h == 0):
                _zero_accs()
            plsc.subcore_barrier()


def _gate_tail(acc, o_ref):
    u = jax.nn.sigmoid(acc[:, :U])
    cc = jnp.tanh(acc[:, U:])
    o_ref[...] = (1.0 - u) * cc


def _tc1_body(x0_ref, xs_ref, w_ref, b_ref, o_ref, o2_ref):
    acc = jnp.zeros((RBLK, 2 * U), jnp.float32) + b_ref[...]
    for m in range(5):
        for d in range(IN):
            col = x0_ref[d] if m == 0 else xs_ref[m - 1, d]
            acc = acc + col[:, None] * w_ref[m, d][None, :]
    u = jax.nn.sigmoid(acc[:, :U])
    cc = jnp.tanh(acc[:, U:])
    h = (1.0 - u) * cc
    o_ref[...] = h
    o2_ref[0] = h[:, :U // 2]
    o2_ref[1] = h[:, U // 2:]


def _tc1_call(x0c, xs1, wstack, bias):
    grid = (B * NPAD) // RBLK
    return pl.pallas_call(
        _tc1_body,
        grid=(grid,),
        in_specs=[
            pl.BlockSpec((IN, RBLK), lambda i: (0, i)),
            pl.BlockSpec((4, IN, RBLK), lambda i: (0, 0, i)),
            pl.BlockSpec((5, IN, 2 * U), lambda i: (0, 0, 0)),
            pl.BlockSpec((1, 2 * U), lambda i: (0, 0)),
        ],
        out_specs=[
            pl.BlockSpec((RBLK, U), lambda i: (i, 0)),
            pl.BlockSpec((2, RBLK, U // 2), lambda i: (0, i, 0)),
        ],
        out_shape=[
            jax.ShapeDtypeStruct((B * NPAD, U), jnp.float32),
            jax.ShapeDtypeStruct((2, B * NPAD, U // 2), jnp.float32),
        ],
    )(x0c, xs1, wstack, bias)


def _tc2_body(x0_ref, xs_ref, w_ref, b_ref, o_ref):
    acc = jnp.dot(x0_ref[...], w_ref[0], preferred_element_type=jnp.float32)
    for m in range(4):
        acc = acc + jnp.dot(xs_ref[m], w_ref[m + 1],
                            preferred_element_type=jnp.float32)
    acc = acc + b_ref[...]
    _gate_tail(acc, o_ref)


def _tc2_call(x0, xs, wstack, bias):
    grid = (B * NPAD) // RBLK
    return pl.pallas_call(
        _tc2_body,
        grid=(grid,),
        in_specs=[
            pl.BlockSpec((RBLK, U), lambda i: (i, 0)),
            pl.BlockSpec((4, RBLK, U), lambda i: (0, i, 0)),
            pl.BlockSpec((5, U, 2 * U), lambda i: (0, 0, 0)),
            pl.BlockSpec((1, 2 * U), lambda i: (0, 0)),
        ],
        out_specs=pl.BlockSpec((RBLK, U), lambda i: (i, 0)),
        out_shape=jax.ShapeDtypeStruct((B * NPAD, U), jnp.float32),
    )(x0, xs, wstack, bias)


def _prep_weights(Wg, bg, Wc, bc, din):
    dfull = Wg.shape[0] // 5
    Wgr = Wg.reshape(dfull, 5, 2 * U)[:din, :, U:]
    Wcr = Wc.reshape(dfull, 5, U)[:din, :, :]
    Wm = jnp.concatenate([Wgr, Wcr], axis=-1)
    Wt = jnp.moveaxis(Wm, 1, 0)
    W0 = Wt[0] - Wt[2] - Wt[4]
    Ws = jnp.stack([W0, Wt[1], 2.0 * Wt[2], Wt[3], 2.0 * Wt[4]], 0)
    bias = jnp.concatenate([bg[U:], bc])[None, :]
    return Ws, bias


def kernel(inputs, edge_index, adj_vals, W_gate1, b_gate1, W_cand1, b_cand1,
           W_gate2, b_gate2, W_cand2, b_cand2):
    src = edge_index[0]
    dst = edge_index[1]
    pe = EPAD - E
    srcp = jnp.concatenate([src, jnp.zeros((pe,), jnp.int32)]).reshape(ER, 128)
    dstp = jnp.concatenate([dst, jnp.zeros((pe,), jnp.int32)]).reshape(ER, 128)
    adjp = jnp.concatenate([adj_vals, jnp.zeros((pe,), jnp.float32)]).reshape(ER, 128)
    x0 = inputs.reshape(B, N, IN)
    x0p = jnp.pad(x0, ((0, 0), (0, NPAD - N), (0, 0)))
    x0c = x0p.transpose(2, 0, 1).reshape(IN, B * NPAD)

    wn1, wn2, xs1 = _l1_kernel(srcp, dstp, adjp, x0c)
    Ws1, b1 = _prep_weights(W_gate1, b_gate1, W_cand1, b_cand1, IN)
    nh1p, nh1h = _tc1_call(x0c, xs1, Ws1, b1)

    xs2 = _l2_kernel(srcp, dstp, wn1, wn2, nh1h)
    Ws2, b2 = _prep_weights(W_gate2, b_gate2, W_cand2, b_cand2, U)
    nh2p = _tc2_call(nh1p, xs2, Ws2, b2)

    def unpad(a):
        return a.reshape(B, NPAD, U)[:, :N, :].reshape(B, N * U)

    h1 = unpad(nh1p)
    h2 = unpad(nh2p)
    return h2, jnp.stack([h1, h2], 0)


# speedup vs baseline: 11.6112x; 1.0482x over previous
import functools

import jax
import jax.numpy as jnp
from jax import lax
from jax.experimental import pallas as pl
from jax.experimental.pallas import tpu as pltpu
from jax.experimental.pallas import tpu_sc as plsc

N = 10000
E = 160000
U = 64
IN = 2
B = 2
NPAD = 10240
EPAD = 163840
ER = 1280
RPT = 80
NPT = 640
RBLK = 2048

_mesh = plsc.VectorSubcoreMesh(core_axis_name="c", subcore_axis_name="s")
_sc_params = pltpu.CompilerParams(needs_layout_passes=False,
                                  use_tc_tiling_on_sc=False)


def _i16(val):
    return jnp.full((16,), val, jnp.int32)


@functools.partial(
    pl.kernel,
    out_type=[
        jax.ShapeDtypeStruct((ER, 128), jnp.float32),
        jax.ShapeDtypeStruct((ER, 128), jnp.float32),
        jax.ShapeDtypeStruct((4, IN, B * NPAD), jnp.float32),
    ],
    mesh=_mesh,
    compiler_params=_sc_params,
    scratch_types=[
        pltpu.VMEM((RPT, 128), jnp.int32),
        pltpu.VMEM((RPT, 128), jnp.int32),
        pltpu.VMEM((RPT, 128), jnp.float32),
        pltpu.VMEM((RPT, 128), jnp.float32),
        pltpu.VMEM((RPT, 128), jnp.float32),
        pltpu.VMEM((NPAD,), jnp.float32),
        pltpu.VMEM((NPAD,), jnp.float32),
        pltpu.VMEM((NPAD,), jnp.float32),
        pltpu.VMEM((NPAD,), jnp.float32),
        pltpu.VMEM((NPAD,), jnp.float32),
        pltpu.VMEM((NPAD,), jnp.float32),
        pltpu.VMEM((128,), jnp.float32),
        pltpu.VMEM((128,), jnp.float32),
        pltpu.VMEM((NPT,), jnp.float32),
        pltpu.VMEM((NPT,), jnp.float32),
        pltpu.VMEM_SHARED((NPAD,), jnp.float32),
        pltpu.VMEM_SHARED((NPAD,), jnp.float32),
        pltpu.VMEM_SHARED((NPAD,), jnp.float32),
        pltpu.VMEM_SHARED((NPAD,), jnp.float32),
        pltpu.SemaphoreType.DMA,
    ],
)
def _l1_kernel(src_h, dst_h, adj_h, x0_h, wn1_h, wn2_h, xs1_h,
               ev_src, ev_dst, ev_adj, ev_w1, ev_w2, inv_s, inv_d,
               xc0, xc1, cur0, cur1, valb0, valb1, zb1, nsl,
               acc_s, acc_d, accL0, accL1, sem):
    c = lax.axis_index("c")
    t = lax.axis_index("s")
    r0 = t * RPT
    n0 = t * NPT
    iota = lax.iota(jnp.int32, 16)
    zv = jnp.zeros((16,), jnp.float32)

    pltpu.sync_copy(src_h.at[pl.ds(r0, RPT)], ev_src)
    pltpu.sync_copy(dst_h.at[pl.ds(r0, RPT)], ev_dst)
    pltpu.sync_copy(adj_h.at[pl.ds(r0, RPT)], ev_adj)
    pltpu.sync_copy(x0_h.at[0].at[pl.ds(c * NPAD, NPAD)], xc0)
    pltpu.sync_copy(x0_h.at[1].at[pl.ds(c * NPAD, NPAD)], xc1)

    @pl.loop(0, NPT // 16)
    def _(k):
        plsc.store_scatter(zb1, [k * 16 + iota], zv)

    for accref in (acc_s, acc_d, accL0, accL1):
        pltpu.sync_copy(zb1, accref.at[pl.ds(n0, NPT)])
    plsc.subcore_barrier()

    @pl.loop(0, RPT // 2)
    def _(i2):
        i0 = i2 * 2
        d0 = pltpu.make_async_copy(ev_adj.at[i0], acc_s.at[ev_src.at[i0]], sem)
        d0.start(add=True)
        d1 = pltpu.make_async_copy(ev_adj.at[i0], acc_d.at[ev_dst.at[i0]], sem)
        d1.start(add=True)
        d2 = pltpu.make_async_copy(ev_adj.at[i0 + 1], acc_s.at[ev_src.at[i0 + 1]], sem)
        d2.start(add=True)
        d3 = pltpu.make_async_copy(ev_adj.at[i0 + 1], acc_d.at[ev_dst.at[i0 + 1]], sem)
        d3.start(add=True)
        d0.wait()
        d1.wait()
        d2.wait()
        d3.wait()
    plsc.subcore_barrier()

    for accref in (acc_s, acc_d):
        pltpu.sync_copy(accref.at[pl.ds(n0, NPT)], nsl)

        @pl.loop(0, NPT // 16)
        def _(k):
            f = k * 16 + iota
            v = plsc.load_gather(nsl, [f])
            v = jnp.where(v > 0.0, v, 1.0)
            plsc.store_scatter(nsl, [f], 1.0 / v)

        pltpu.sync_copy(nsl, accref.at[pl.ds(n0, NPT)])
    plsc.subcore_barrier()
    pltpu.sync_copy(acc_s, inv_s)
    pltpu.sync_copy(acc_d, inv_d)

    @pl.loop(0, RPT)
    def _(i):
        ri = _i16(i)

        @pl.loop(0, 8)
        def _(k):
            col = k * 16 + iota
            av = plsc.load_gather(ev_adj, [ri, col])
            sv = plsc.load_gather(ev_src, [ri, col])
            dv = plsc.load_gather(ev_dst, [ri, col])
            plsc.store_scatter(ev_w1, [ri, col], av * plsc.load_gather(inv_s, [sv]))
            plsc.store_scatter(ev_w2, [ri, col], av * plsc.load_gather(inv_d, [dv]))

    @pl.when(c == 0)
    def _():
        pltpu.sync_copy(ev_w1, wn1_h.at[pl.ds(r0, RPT)])
        pltpu.sync_copy(ev_w2, wn2_h.at[pl.ds(r0, RPT)])

    for s in range(4):
        g0 = xc0 if s in (0, 2) else cur0
        g1 = xc1 if s in (0, 2) else cur1
        idxg = ev_src if s < 2 else ev_dst
        idxs = ev_dst if s < 2 else ev_src
        wb = ev_w1 if s < 2 else ev_w2

        @pl.loop(0, RPT)
        def _(i, idxg=idxg, idxs=idxs, wb=wb, g0=g0, g1=g1):
            ri = _i16(i)

            @pl.loop(0, 8)
            def _(k, idxg=idxg, wb=wb, g0=g0, g1=g1, ri=ri):
                col = k * 16 + iota
                sv = plsc.load_gather(idxg, [ri, col])
                wv = plsc.load_gather(wb, [ri, col])
                plsc.store_scatter(valb0, [col], wv * plsc.load_gather(g0, [sv]))
                plsc.store_scatter(valb1, [col], wv * plsc.load_gather(g1, [sv]))

            da = pltpu.make_async_copy(valb0, accL0.at[idxs.at[i]], sem)
            da.start(add=True)
            db = pltpu.make_async_copy(valb1, accL1.at[idxs.at[i]], sem)
            db.start(add=True)
            da.wait()
            db.wait()

        plsc.subcore_barrier()
        if s in (0, 2):
            pltpu.sync_copy(accL0, cur0)
            pltpu.sync_copy(accL1, cur1)
        for d, accref in ((0, accL0), (1, accL1)):
            pltpu.sync_copy(accref.at[pl.ds(n0, NPT)], nsl)
            pltpu.sync_copy(nsl, xs1_h.at[s].at[d].at[pl.ds(c * NPAD + n0, NPT)])
        plsc.subcore_barrier()
        if s < 3:
            pltpu.sync_copy(zb1, accL0.at[pl.ds(n0, NPT)])
            pltpu.sync_copy(zb1, accL1.at[pl.ds(n0, NPT)])
            plsc.subcore_barrier()


UH = U // 2


@functools.partial(
    pl.kernel,
    out_type=jax.ShapeDtypeStruct((4, B * NPAD, U), jnp.float32),
    mesh=_mesh,
    compiler_params=_sc_params,
    scratch_types=[
        pltpu.VMEM((RPT, 128), jnp.int32),
        pltpu.VMEM((RPT, 128), jnp.int32),
        pltpu.VMEM((RPT, 128), jnp.float32),
        pltpu.VMEM((RPT, 128), jnp.float32),
        [pltpu.VMEM((128, UH), jnp.float32) for _ in range(4)],
        [pltpu.VMEM((128, UH), jnp.float32) for _ in range(2)],
        pltpu.VMEM_SHARED((NPAD, UH), jnp.float32),
        pltpu.VMEM_SHARED((NPAD, UH), jnp.float32),
        pltpu.SemaphoreType.DMA,
        pltpu.SemaphoreType.DMA,
    ],
)
def _l2_kernel(src_h, dst_h, wn1_h, wn2_h, x2_h, xs2_h,
               ev_src, ev_dst, ev_w1, ev_w2, gbs, sbs, accA, accB,
               sem_g, sem_s):
    c = lax.axis_index("c")
    t = lax.axis_index("s")
    r0 = t * RPT
    n0 = t * NPT
    off = c * NPAD
    iota = lax.iota(jnp.int32, 16)
    zv = jnp.zeros((16,), jnp.float32)

    def _fill_zero(gb):
        @pl.loop(0, 128 * UH // 16)
        def _(k, gb=gb):
            f = k * 16 + iota
            plsc.store_scatter(gb, [f // UH, f % UH], zv)

    def _zero_accs():
        _fill_zero(gbs[0])
        for accref in (accA, accB):
            for kc in range(NPT // 128):
                pltpu.sync_copy(gbs[0], accref.at[pl.ds(n0 + kc * 128, 128)])

    pltpu.sync_copy(src_h.at[pl.ds(r0, RPT)], ev_src)
    pltpu.sync_copy(dst_h.at[pl.ds(r0, RPT)], ev_dst)
    pltpu.sync_copy(wn1_h.at[pl.ds(r0, RPT)], ev_w1)
    pltpu.sync_copy(wn2_h.at[pl.ds(r0, RPT)], ev_w2)

    _zero_accs()
    plsc.subcore_barrier()

    for h in range(2):
        for s in range(4):
            hbm_src = s in (0, 2)
            gsrc = x2_h.at[h].at[pl.ds(off, NPAD)] if hbm_src else accA
            acc = accA if hbm_src else accB
            idxg = ev_src if s < 2 else ev_dst
            idxs = ev_dst if s < 2 else ev_src
            wb = ev_w1 if s < 2 else ev_w2

            def _gather(i, gb, gsrc=gsrc, idxg=idxg):
                return pltpu.make_async_copy(gsrc.at[idxg.at[i]], gb, sem_g)

            def _scatter(i, sb, idxs=idxs, acc=acc):
                return pltpu.make_async_copy(sb, acc.at[idxs.at[i]], sem_s)

            for b in range(2):
                _gather(b, gbs[b]).start()

            @pl.loop(0, RPT // 4)
            def _(i4, wb=wb, _gather=_gather, _scatter=_scatter):
                for b in range(4):
                    i = i4 * 4 + b
                    gb = gbs[b]
                    sb = sbs[b % 2]
                    _gather(i, gb).wait()

                    prev = i - 2

                    @pl.when(prev >= 0)
                    def _(prev=prev, sb=sb, _scatter=_scatter):
                        _scatter(prev, sb).wait()

                    @pl.loop(0, 128, unroll=16)
                    def _(e, gb=gb, sb=sb, wb=wb, i=i):
                        re = _i16(e)
                        wv = plsc.load_gather(wb, [_i16(i), re])
                        for j in range(UH // 16):
                            col = j * 16 + iota
                            v = plsc.load_gather(gb, [re, col])
                            plsc.store_scatter(sb, [re, col], v * wv)

                    _scatter(i, sb).start(add=True)
                    nxt = i + 2

                    @pl.when(nxt < RPT)
                    def _(nxt=nxt, nb=(b + 2) % 4, _gather=_gather):
                        _gather(nxt, gbs[nb]).start()

            _scatter(RPT - 2, sbs[0]).wait()
            _scatter(RPT - 1, sbs[1]).wait()
            plsc.subcore_barrier()
            pltpu.sync_copy(
                acc.at[pl.ds(n0, NPT)],
                xs2_h.at[s].at[pl.ds(off + n0, NPT), pl.ds(h * UH, UH)])
            if s == 1 or (s == 3 and h == 0):
                _zero_accs()
            plsc.subcore_barrier()


def _gate_tail(acc, o_ref):
    u = jax.nn.sigmoid(acc[:, :U])
    cc = jnp.tanh(acc[:, U:])
    o_ref[...] = (1.0 - u) * cc


def _tc1_body(x0_ref, xs_ref, w_ref, b_ref, o_ref, o2_ref):
    acc = jnp.zeros((RBLK, 2 * U), jnp.float32) + b_ref[...]
    for m in range(5):
        for d in range(IN):
            col = x0_ref[d] if m == 0 else xs_ref[m - 1, d]
            acc = acc + col[:, None] * w_ref[m, d][None, :]
    u = jax.nn.sigmoid(acc[:, :U])
    cc = jnp.tanh(acc[:, U:])
    h = (1.0 - u) * cc
    o_ref[...] = h
    o2_ref[0] = h[:, :U // 2]
    o2_ref[1] = h[:, U // 2:]


def _tc1_call(x0c, xs1, wstack, bias):
    grid = (B * NPAD) // RBLK
    return pl.pallas_call(
        _tc1_body,
        grid=(grid,),
        in_specs=[
            pl.BlockSpec((IN, RBLK), lambda i: (0, i)),
            pl.BlockSpec((4, IN, RBLK), lambda i: (0, 0, i)),
            pl.BlockSpec((5, IN, 2 * U), lambda i: (0, 0, 0)),
            pl.BlockSpec((1, 2 * U), lambda i: (0, 0)),
        ],
        out_specs=[
            pl.BlockSpec((RBLK, U), lambda i: (i, 0)),
            pl.BlockSpec((2, RBLK, U // 2), lambda i: (0, i, 0)),
        ],
        out_shape=[
            jax.ShapeDtypeStruct((B * NPAD, U), jnp.float32),
            jax.ShapeDtypeStruct((2, B * NPAD, U // 2), jnp.float32),
        ],
    )(x0c, xs1, wstack, bias)


def _tc2_body(x0_ref, xs_ref, w_ref, b_ref, o_ref):
    acc = jnp.dot(x0_ref[...], w_ref[0], preferred_element_type=jnp.float32)
    for m in range(4):
        acc = acc + jnp.dot(xs_ref[m], w_ref[m + 1],
                            preferred_element_type=jnp.float32)
    acc = acc + b_ref[...]
    _gate_tail(acc, o_ref)


def _tc2_call(x0, xs, wstack, bias):
    grid = (B * NPAD) // RBLK
    return pl.pallas_call(
        _tc2_body,
        grid=(grid,),
        in_specs=[
            pl.BlockSpec((RBLK, U), lambda i: (i, 0)),
            pl.BlockSpec((4, RBLK, U), lambda i: (0, i, 0)),
            pl.BlockSpec((5, U, 2 * U), lambda i: (0, 0, 0)),
            pl.BlockSpec((1, 2 * U), lambda i: (0, 0)),
        ],
        out_specs=pl.BlockSpec((RBLK, U), lambda i: (i, 0)),
        out_shape=jax.ShapeDtypeStruct((B * NPAD, U), jnp.float32),
    )(x0, xs, wstack, bias)


def _prep_weights(Wg, bg, Wc, bc, din):
    dfull = Wg.shape[0] // 5
    Wgr = Wg.reshape(dfull, 5, 2 * U)[:din, :, U:]
    Wcr = Wc.reshape(dfull, 5, U)[:din, :, :]
    Wm = jnp.concatenate([Wgr, Wcr], axis=-1)
    Wt = jnp.moveaxis(Wm, 1, 0)
    W0 = Wt[0] - Wt[2] - Wt[4]
    Ws = jnp.stack([W0, Wt[1], 2.0 * Wt[2], Wt[3], 2.0 * Wt[4]], 0)
    bias = jnp.concatenate([bg[U:], bc])[None, :]
    return Ws, bias


def kernel(inputs, edge_index, adj_vals, W_gate1, b_gate1, W_cand1, b_cand1,
           W_gate2, b_gate2, W_cand2, b_cand2):
    src = edge_index[0]
    dst = edge_index[1]
    pe = EPAD - E
    srcp = jnp.concatenate([src, jnp.zeros((pe,), jnp.int32)]).reshape(ER, 128)
    dstp = jnp.concatenate([dst, jnp.zeros((pe,), jnp.int32)]).reshape(ER, 128)
    adjp = jnp.concatenate([adj_vals, jnp.zeros((pe,), jnp.float32)]).reshape(ER, 128)
    x0 = inputs.reshape(B, N, IN)
    x0p = jnp.pad(x0, ((0, 0), (0, NPAD - N), (0, 0)))
    x0c = x0p.transpose(2, 0, 1).reshape(IN, B * NPAD)

    wn1, wn2, xs1 = _l1_kernel(srcp, dstp, adjp, x0c)
    Ws1, b1 = _prep_weights(W_gate1, b_gate1, W_cand1, b_cand1, IN)
    nh1p, nh1h = _tc1_call(x0c, xs1, Ws1, b1)

    xs2 = _l2_kernel(srcp, dstp, wn1, wn2, nh1h)
    Ws2, b2 = _prep_weights(W_gate2, b_gate2, W_cand2, b_cand2, U)
    nh2p = _tc2_call(nh1p, xs2, Ws2, b2)

    def unpad(a):
        return a.reshape(B, NPAD, U)[:, :N, :].reshape(B, N * U)

    h1 = unpad(nh1p)
    h2 = unpad(nh2p)
    return h2, jnp.stack([h1, h2], 0)
